# baseline (speedup 1.0000x reference)
import functools

import jax
import jax.numpy as jnp
from jax.experimental import pallas as pl
from jax.experimental.pallas import tpu as pltpu


def _leaky(x):
    return jnp.where(x >= 0, x, 0.01 * x)


def _bf_dot(a, b):
    return jax.lax.dot_general(
        a, b, (((1,), (0,)), ((), ())),
        preferred_element_type=jnp.float32)


def _dot(a, b):
    return _bf_dot(a.astype(jnp.bfloat16), b.astype(jnp.bfloat16))


def _fused_kernel(obs_ref, adj_ref, w1_ref, b1_ref, w2_ref, b2_ref,
                  msgw_ref, msgb_ref, wih_ref, whh_ref, bih_ref, bhh_ref,
                  headw_ref, headb_ref, regw_ref, regb_ref,
                  regallw_ref, regallb_ref,
                  cls_ref, pred_ref, predall_ref, state_scr,
                  *, iterations, row_tile, head_tile):
    p = pl.program_id(1)
    num_nodes = adj_ref.shape[1]

    @pl.when(p == 0)
    def _gnn_phase():
        obs = obs_ref[0]
        h = _leaky(_dot(obs, w1_ref[...]) + b1_ref[...])
        state = _leaky(_dot(h, w2_ref[...]) + b2_ref[...])
        d = state.shape[1]
        msg_ws = msgw_ref[:d, :]
        msg_wa = msgw_ref[d:, :]
        for _ in range(iterations):
            s_hi = state.astype(jnp.bfloat16)
            tiles = []
            for t in range(num_nodes // row_tile):
                adj_t = adj_ref[pl.ds(0, 1), pl.ds(t * row_tile, row_tile),
                                :][0].astype(jnp.bfloat16)
                tiles.append(_bf_dot(adj_t, s_hi))
            agg = jnp.concatenate(tiles, axis=0)
            m = _leaky(_dot(state, msg_ws) + _dot(agg, msg_wa)
                       + msgb_ref[...])
            gi = _dot(m, wih_ref[...]) + bih_ref[...]
            gh = _dot(state, whh_ref[...]) + bhh_ref[...]
            i_r, i_z, i_n = gi[:, :d], gi[:, d:2 * d], gi[:, 2 * d:]
            h_r, h_z, h_n = gh[:, :d], gh[:, d:2 * d], gh[:, 2 * d:]
            r = jax.nn.sigmoid(i_r + h_r)
            z = jax.nn.sigmoid(i_z + h_z)
            n = jnp.tanh(i_n + r * h_n)
            state = (1.0 - z) * n + z * state
        state_scr[...] = state
        cls_ref[0] = _dot(state, headw_ref[...]) + headb_ref[...]
        pred_ref[0] = _dot(state, regw_ref[...]) + regb_ref[...]

    @pl.when(p > 0)
    def _head_phase():
        row0 = (p - 1) * head_tile
        s = state_scr[pl.ds(row0, head_tile), :]
        predall_ref[0] = _dot(s, regallw_ref[...]) + regallb_ref[...]


def kernel(node_obs, node_adj, enc_W1, enc_b1, enc_W2, enc_b2, msg_W, msg_b,
           W_ih, W_hh, b_ih, b_hh, head_W, head_b, reg_W, reg_b,
           regall_W, regall_b):
    B, N, F_in = node_obs.shape
    D = enc_W2.shape[1]
    ENC = enc_W1.shape[1]
    NB_CLASSES = head_W.shape[1]
    NB_NODES = regall_W.shape[1]
    HEAD_TILE = 256
    n_head_tiles = N // HEAD_TILE

    row2 = lambda v: v.reshape(1, -1)
    const2 = lambda shape: pl.BlockSpec(shape, lambda b, p: (0, 0))

    cls, pred, pred_all = pl.pallas_call(
        functools.partial(_fused_kernel, iterations=3, row_tile=256,
                          head_tile=HEAD_TILE),
        grid=(B, 1 + n_head_tiles),
        in_specs=[
            pl.BlockSpec((1, N, F_in), lambda b, p: (b, 0, 0)),
            pl.BlockSpec((1, N, N), lambda b, p: (b, 0, 0)),
            const2((F_in, ENC)),
            const2((1, ENC)),
            const2((ENC, D)),
            const2((1, D)),
            const2((2 * D, D)),
            const2((1, D)),
            const2((D, 3 * D)),
            const2((D, 3 * D)),
            const2((1, 3 * D)),
            const2((1, 3 * D)),
            const2((D, NB_CLASSES)),
            const2((1, NB_CLASSES)),
            const2((D, 1)),
            const2((1, 1)),
            const2((D, NB_NODES)),
            const2((1, NB_NODES)),
        ],
        out_specs=[
            pl.BlockSpec((1, N, NB_CLASSES), lambda b, p: (b, 0, 0)),
            pl.BlockSpec((1, N, 1), lambda b, p: (b, 0, 0)),
            pl.BlockSpec((1, HEAD_TILE, NB_NODES),
                         lambda b, p: (b, jnp.maximum(p - 1, 0), 0)),
        ],
        out_shape=[
            jax.ShapeDtypeStruct((B, N, NB_CLASSES), jnp.float32),
            jax.ShapeDtypeStruct((B, N, 1), jnp.float32),
            jax.ShapeDtypeStruct((B, N, NB_NODES), jnp.float32),
        ],
        scratch_shapes=[pltpu.VMEM((N, D), jnp.float32)],
    )(node_obs, node_adj, enc_W1, row2(enc_b1), enc_W2, row2(enc_b2),
      msg_W, row2(msg_b), W_ih, W_hh, row2(b_ih), row2(b_hh),
      head_W, row2(head_b), reg_W, row2(reg_b), regall_W, row2(regall_b))

    return (cls, pred, pred_all)
